# Initial kernel scaffold; baseline (speedup 1.0000x reference)
import jax
import jax.numpy as jnp
from jax.experimental import pallas as pl


def kernel(logits, labels, attention_weights, lengths, params):
    raise NotImplementedError("write your pallas kernel here")


# speedup vs baseline: 1.0976x; 1.0976x over previous
import functools
import math

import jax
import jax.numpy as jnp
from jax.experimental import pallas as pl
from jax.experimental.pallas import tpu as pltpu

ALPHA = 1e-4
BETA = 1.0
MAX_STD = 1000.0
MIN_STD = 1.0

_INV_SQRT_2PI = 1.0 / math.sqrt(2.0 * math.pi)


def _fused_body(logits_ref, labels_ref, aw_ref, len_ref, params_ref,
                loss_ref, nll_ref):
    logits = logits_ref[...]
    m = jnp.max(logits, axis=1, keepdims=True)
    lse = m + jnp.log(jnp.sum(jnp.exp(logits - m), axis=1, keepdims=True))
    logp = logits - lse
    labels = labels_ref[...]
    picked = jnp.where(labels == 1, logp[:, 1:2], logp[:, 0:1])
    nll = -jnp.mean(picked)

    aw = aw_ref[...]
    b, seg_len = aw.shape
    idx = jax.lax.broadcasted_iota(jnp.int32, (b, seg_len), 1)
    x = (idx.astype(jnp.float32) + 1.0) / seg_len
    sums = jnp.sum(aw, axis=1, keepdims=True)
    means = jnp.sum(x * aw, axis=1, keepdims=True) / sums
    len_f = len_ref[...].astype(jnp.float32)
    ideal_stds = jnp.where(labels == 1, MIN_STD / len_f, MAX_STD / len_f)
    z = (x - means) / ideal_stds
    r_hats = jnp.exp(-0.5 * z * z) * (_INV_SQRT_2PI / ideal_stds)
    rs = r_hats / (jnp.sum(r_hats, axis=1, keepdims=True) + 1e-6)
    diff = aw - rs
    aw_penalty = (BETA / 2.0) * jnp.mean(diff * diff)

    p = params_ref[...]
    penalty = (ALPHA / 2.0) * jnp.sum(jnp.abs(p))

    nll_ref[...] = jnp.reshape(nll, (1, 1))
    loss_ref[...] = jnp.reshape(nll + penalty + aw_penalty, (1, 1))


@functools.partial(jax.jit, static_argnames=())
def _run(logits, labels2d, aw2d, lengths2d, params2d):
    out = pl.pallas_call(
        _fused_body,
        out_shape=(
            jax.ShapeDtypeStruct((1, 1), jnp.float32),
            jax.ShapeDtypeStruct((1, 1), jnp.float32),
        ),
    )(logits, labels2d, aw2d, lengths2d, params2d)
    return out


def kernel(logits, labels, attention_weights, lengths, params):
    b = lengths.shape[0]
    seg_len = attention_weights.shape[0] // b
    aw2d = attention_weights.reshape(b, seg_len)
    labels2d = labels.astype(jnp.int32).reshape(b, 1)
    lengths2d = lengths.reshape(b, 1)
    params2d = params.reshape(-1, 128)
    loss, nll = _run(logits, labels2d, aw2d, lengths2d, params2d)
    return (loss[0, 0], nll[0, 0])
